# baseline (device time: 15404 ns/iter reference)
import jax
import jax.numpy as jnp
from jax import lax
from jax.experimental import pallas as pl
from jax.experimental.pallas import tpu as pltpu

import os

N_CHUNKS = int(os.environ.get("AG_CHUNKS", "8"))
AG_VARIANT = int(os.environ.get("AG_VARIANT", "0"))
AG_ROWS = int(os.environ.get("AG_ROWS", "256"))


def kernel(x):
    m, n = x.shape
    half = m // 2
    ch = half // N_CHUNKS

    def body(x_ref, out_ref, p1_send, p1_recv, p2_send, p2_recv, z_sem):
        my_x = lax.axis_index("x")
        my_y = lax.axis_index("y")
        my_z = lax.axis_index("z")
        nbr_y = (my_x, 1 - my_y, my_z)
        nbr_z = (my_x, my_y, 1 - my_z)

        if AG_VARIANT == 6:
            out_ref[pl.ds(0, 8), :] = x_ref[pl.ds(0, 8), :]
            return

        barrier = pltpu.get_barrier_semaphore()
        pl.semaphore_signal(
            barrier, inc=1, device_id=nbr_y,
            device_id_type=pl.DeviceIdType.MESH,
        )
        pl.semaphore_signal(
            z_sem, inc=1, device_id=nbr_z,
            device_id_type=pl.DeviceIdType.MESH,
        )
        pl.semaphore_wait(barrier, 1)

        my_off = my_y * m
        miss_off = (1 - my_y) * m
        zh = my_z * half

        if AG_VARIANT == 1:
            out_ref[pl.ds(my_off, m), :] = x_ref[:, :]
            out_ref[pl.ds(miss_off, m), :] = x_ref[:, :]
            return

        if AG_VARIANT == 5:
            return

        if AG_VARIANT in (3, 4):
            chv = AG_ROWS // N_CHUNKS
            def _send_all():
                descs = []
                for c in range(N_CHUNKS):
                    s = pltpu.make_async_remote_copy(
                        src_ref=x_ref.at[pl.ds(c * chv, chv)],
                        dst_ref=out_ref.at[pl.ds(my_off + c * chv, chv)],
                        send_sem=p1_send.at[c],
                        recv_sem=p1_recv.at[c],
                        device_id=nbr_y,
                        device_id_type=pl.DeviceIdType.MESH,
                    )
                    s.start()
                    descs.append(s)
                for s in descs:
                    s.wait_send()

            def _recv_all():
                for c in range(N_CHUNKS):
                    r = pltpu.make_async_remote_copy(
                        src_ref=x_ref.at[pl.ds(0, chv)],
                        dst_ref=out_ref.at[pl.ds(miss_off + c * chv, chv)],
                        send_sem=p1_send.at[c],
                        recv_sem=p1_recv.at[c],
                        device_id=nbr_y,
                        device_id_type=pl.DeviceIdType.MESH,
                    )
                    r.wait_recv()

            if AG_VARIANT == 3:
                pl.when(my_y == 0)(_send_all)
                pl.when(my_y == 1)(_recv_all)
            else:
                _send_all()
                _recv_all()
            return

        sends1 = []
        for c in range(N_CHUNKS):
            s = pltpu.make_async_remote_copy(
                src_ref=x_ref.at[pl.ds(zh + c * ch, ch)],
                dst_ref=out_ref.at[pl.ds(my_off + zh + c * ch, ch)],
                send_sem=p1_send.at[c],
                recv_sem=p1_recv.at[c],
                device_id=nbr_y,
                device_id_type=pl.DeviceIdType.MESH,
            )
            s.start()
            sends1.append(s)

        out_ref[pl.ds(my_off, m), :] = x_ref[:, :]

        pl.semaphore_wait(z_sem, 1)

        sends2 = []
        for c in range(N_CHUNKS):
            rows = pl.ds(miss_off + zh + c * ch, ch)
            recv = pltpu.make_async_remote_copy(
                src_ref=x_ref.at[pl.ds(0, ch)],
                dst_ref=out_ref.at[rows],
                send_sem=p1_send.at[c],
                recv_sem=p1_recv.at[c],
                device_id=nbr_y,
                device_id_type=pl.DeviceIdType.MESH,
            )
            recv.wait_recv()
            if AG_VARIANT == 2:
                continue
            s = pltpu.make_async_remote_copy(
                src_ref=out_ref.at[rows],
                dst_ref=out_ref.at[rows],
                send_sem=p2_send.at[c],
                recv_sem=p2_recv.at[c],
                device_id=nbr_z,
                device_id_type=pl.DeviceIdType.MESH,
            )
            s.start()
            sends2.append(s)

        z2h = (1 - my_z) * half
        for c in range(N_CHUNKS if AG_VARIANT == 0 else 0):
            recv = pltpu.make_async_remote_copy(
                src_ref=x_ref.at[pl.ds(0, ch)],
                dst_ref=out_ref.at[pl.ds(miss_off + z2h + c * ch, ch)],
                send_sem=p2_send.at[c],
                recv_sem=p2_recv.at[c],
                device_id=nbr_z,
                device_id_type=pl.DeviceIdType.MESH,
            )
            recv.wait_recv()

        for s in sends1 + sends2:
            s.wait_send()

    out_shape = jax.ShapeDtypeStruct((2 * m, n), x.dtype)
    return pl.pallas_call(
        body,
        out_shape=out_shape,
        in_specs=[pl.BlockSpec(memory_space=pltpu.VMEM)],
        out_specs=pl.BlockSpec(memory_space=pltpu.VMEM),
        scratch_shapes=[
            pltpu.SemaphoreType.DMA((N_CHUNKS,)),
            pltpu.SemaphoreType.DMA((N_CHUNKS,)),
            pltpu.SemaphoreType.DMA((N_CHUNKS,)),
            pltpu.SemaphoreType.DMA((N_CHUNKS,)),
            pltpu.SemaphoreType.REGULAR,
        ],
        compiler_params=(
            pltpu.CompilerParams()
            if AG_VARIANT == 6
            else pltpu.CompilerParams(collective_id=0)
        ),
    )(x)


# device time: 13917 ns/iter; 1.1068x vs baseline; 1.1068x over previous
import jax
import jax.numpy as jnp
from jax import lax
from jax.experimental import pallas as pl
from jax.experimental.pallas import tpu as pltpu

import os

D_ROWS = int(os.environ.get("AG_D", "320"))
N_FWD = 8
N_PRIV = 2


def kernel(x):
    m, n = x.shape
    d = D_ROWS
    fwd = m - d
    fc = fwd // N_FWD
    priv = 2 * d - m
    pc = priv // N_PRIV
    assert fwd % N_FWD == 0 and priv % N_PRIV == 0
    NT = N_FWD + N_PRIV

    def body(x_ref, out_ref, p1_send, p1_recv, p2_send, p2_recv,
             x_sem, z_sem):
        my_x = lax.axis_index("x")
        my_y = lax.axis_index("y")
        my_z = lax.axis_index("z")
        nbr_y = (my_x, 1 - my_y, my_z)
        nbr_x = (1 - my_x, my_y, my_z)
        nbr_z = (my_x, my_y, 1 - my_z)
        hh = lax.rem(my_x + my_z, 2)

        barrier = pltpu.get_barrier_semaphore()
        pl.semaphore_signal(
            barrier, inc=1, device_id=nbr_y,
            device_id_type=pl.DeviceIdType.MESH,
        )
        pl.semaphore_signal(
            x_sem, inc=1, device_id=nbr_x,
            device_id_type=pl.DeviceIdType.MESH,
        )
        pl.semaphore_signal(
            z_sem, inc=1, device_id=nbr_z,
            device_id_type=pl.DeviceIdType.MESH,
        )
        pl.semaphore_wait(barrier, 1)

        my_off = my_y * m
        miss_off = (1 - my_y) * m

        fs_base = hh * d
        pv_base = fwd
        comp_base = (1 - hh) * d

        chunks = [(fs_base + c * fc, fc) for c in range(N_FWD)]
        chunks += [(pv_base + c * pc, pc) for c in range(N_PRIV)]

        sends1 = []
        for i, (row, nr) in enumerate(chunks):
            s = pltpu.make_async_remote_copy(
                src_ref=x_ref.at[pl.ds(row, nr)],
                dst_ref=out_ref.at[pl.ds(my_off + row, nr)],
                send_sem=p1_send.at[i],
                recv_sem=p1_recv.at[i],
                device_id=nbr_y,
                device_id_type=pl.DeviceIdType.MESH,
            )
            s.start()
            sends1.append(s)

        out_ref[pl.ds(my_off, m), :] = x_ref[:, :]

        pl.semaphore_wait(x_sem, 1)
        pl.semaphore_wait(z_sem, 1)

        sends2 = []
        for i, (row, nr) in enumerate(chunks):
            rows = pl.ds(miss_off + row, nr)
            recv = pltpu.make_async_remote_copy(
                src_ref=x_ref.at[pl.ds(0, nr)],
                dst_ref=out_ref.at[rows],
                send_sem=p1_send.at[i],
                recv_sem=p1_recv.at[i],
                device_id=nbr_y,
                device_id_type=pl.DeviceIdType.MESH,
            )
            recv.wait_recv()
            if i >= N_FWD:
                continue
            tgt = nbr_x if i % 2 == 0 else nbr_z
            s = pltpu.make_async_remote_copy(
                src_ref=out_ref.at[rows],
                dst_ref=out_ref.at[rows],
                send_sem=p2_send.at[i],
                recv_sem=p2_recv.at[i],
                device_id=tgt,
                device_id_type=pl.DeviceIdType.MESH,
            )
            s.start()
            sends2.append(s)

        for c in range(N_FWD):
            recv = pltpu.make_async_remote_copy(
                src_ref=x_ref.at[pl.ds(0, fc)],
                dst_ref=out_ref.at[pl.ds(miss_off + comp_base + c * fc, fc)],
                send_sem=p2_send.at[c],
                recv_sem=p2_recv.at[c],
                device_id=nbr_x if c % 2 == 0 else nbr_z,
                device_id_type=pl.DeviceIdType.MESH,
            )
            recv.wait_recv()

        for s in sends1 + sends2:
            s.wait_send()

    out_shape = jax.ShapeDtypeStruct((2 * m, n), x.dtype)
    return pl.pallas_call(
        body,
        out_shape=out_shape,
        in_specs=[pl.BlockSpec(memory_space=pltpu.VMEM)],
        out_specs=pl.BlockSpec(memory_space=pltpu.VMEM),
        scratch_shapes=[
            pltpu.SemaphoreType.DMA((NT,)),
            pltpu.SemaphoreType.DMA((NT,)),
            pltpu.SemaphoreType.DMA((N_FWD,)),
            pltpu.SemaphoreType.DMA((N_FWD,)),
            pltpu.SemaphoreType.REGULAR,
            pltpu.SemaphoreType.REGULAR,
        ],
        compiler_params=pltpu.CompilerParams(collective_id=0),
    )(x)
